# baseline (speedup 1.0000x reference)
import functools

import jax
import jax.numpy as jnp
from jax import lax
from jax.experimental import pallas as pl
from jax.experimental.pallas import tpu as pltpu
from jax.experimental.pallas import tpu_sc as plsc

N = 100000
E = 1600000
E8 = E // 8
NC = 2
NS = 16
NW = NC * NS
EPW = E // NW
CHUNK = 1000
NCHUNK = EPW // CHUNK
STRIPE = 6256
LAST_STRIPE = N - 15 * STRIPE

_mesh = plsc.VectorSubcoreMesh(core_axis_name="c", subcore_axis_name="s")
_sc_params = pltpu.CompilerParams(use_tc_tiling_on_sc=False)


def _wid():
    return lax.axis_index("s") * NC + lax.axis_index("c")


@jax.jit
def _sc_gather_pq(P, Q, dst, src):
    @functools.partial(
        pl.kernel,
        out_type=(
            jax.ShapeDtypeStruct((E, 16), jnp.float32),
            jax.ShapeDtypeStruct((E, 16), jnp.float32),
        ),
        mesh=_mesh,
        compiler_params=_sc_params,
        scratch_types=[
            pltpu.VMEM((CHUNK,), jnp.int32),
            pltpu.VMEM((CHUNK,), jnp.int32),
            pltpu.VMEM((CHUNK, 16), jnp.float32),
            pltpu.VMEM((CHUNK, 16), jnp.float32),
        ],
    )
    def k(P_hbm, Q_hbm, dst_hbm, src_hbm, gp_hbm, gq_hbm,
          idxd_v, idxs_v, rp_v, rq_v):
        base = _wid() * EPW

        @pl.loop(0, NCHUNK)
        def _(c):
            off = base + c * CHUNK
            pltpu.sync_copy(dst_hbm.at[pl.ds(off, CHUNK)], idxd_v)
            pltpu.sync_copy(src_hbm.at[pl.ds(off, CHUNK)], idxs_v)
            pltpu.sync_copy(P_hbm.at[idxd_v], rp_v)
            pltpu.sync_copy(Q_hbm.at[idxs_v], rq_v)
            pltpu.sync_copy(rp_v, gp_hbm.at[pl.ds(off, CHUNK)])
            pltpu.sync_copy(rq_v, gq_hbm.at[pl.ds(off, CHUNK)])

    return k(P, Q, dst, src)


@jax.jit
def _sc_scatter_add(m, dst):
    @functools.partial(
        pl.kernel,
        out_type=jax.ShapeDtypeStruct((NC, N, 16), jnp.float32),
        mesh=_mesh,
        compiler_params=_sc_params,
        scratch_types=[
            pltpu.VMEM((CHUNK,), jnp.int32),
            pltpu.VMEM((CHUNK, 16), jnp.float32),
            pltpu.VMEM_SHARED((N, 16), jnp.float32),
        ],
    )
    def k(m_hbm, dst_hbm, out_hbm, idx_v, rows_v, acc_sh):
        cid = lax.axis_index("c")
        sid = lax.axis_index("s")
        base = _wid() * EPW

        s0 = sid * STRIPE

        @pl.loop(0, CHUNK)
        def _(i):
            rows_v[i, :] = jnp.zeros((16,), jnp.float32)

        @pl.when(sid < 15)
        def _():
            for j in range(STRIPE // CHUNK):
                pltpu.sync_copy(rows_v,
                                acc_sh.at[pl.ds(s0 + j * CHUNK, CHUNK)])
            pltpu.sync_copy(
                rows_v.at[pl.ds(0, STRIPE % CHUNK)],
                acc_sh.at[pl.ds(s0 + (STRIPE // CHUNK) * CHUNK,
                                STRIPE % CHUNK)])

        @pl.when(sid == 15)
        def _():
            for j in range(LAST_STRIPE // CHUNK):
                pltpu.sync_copy(rows_v,
                                acc_sh.at[pl.ds(s0 + j * CHUNK, CHUNK)])
            pltpu.sync_copy(
                rows_v.at[pl.ds(0, LAST_STRIPE % CHUNK)],
                acc_sh.at[pl.ds(s0 + (LAST_STRIPE // CHUNK) * CHUNK,
                                LAST_STRIPE % CHUNK)])

        plsc.subcore_barrier()

        @pl.loop(0, NCHUNK)
        def _(c):
            off = base + c * CHUNK
            pltpu.sync_copy(dst_hbm.at[pl.ds(off, CHUNK)], idx_v)
            pltpu.sync_copy(m_hbm.at[pl.ds(off, CHUNK)], rows_v)
            pltpu.sync_copy(rows_v, acc_sh.at[idx_v], add=True)

        plsc.subcore_barrier()

        @pl.when(sid < 15)
        def _():
            pltpu.sync_copy(acc_sh.at[pl.ds(s0, STRIPE)],
                            out_hbm.at[cid, pl.ds(s0, STRIPE)])

        @pl.when(sid == 15)
        def _():
            pltpu.sync_copy(acc_sh.at[pl.ds(s0, LAST_STRIPE)],
                            out_hbm.at[cid, pl.ds(s0, LAST_STRIPE)])

    return k(m, dst)


NB = 800
EB8 = 2000


def _tc_pre_body(x_ref, Win_ref, bin_ref, W1_ref, b1_ref, P_ref, Q_ref):
    X = x_ref[...]
    H = jnp.tanh(jnp.dot(X, Win_ref[...],
                         preferred_element_type=jnp.float32) + bin_ref[...])
    h = jnp.concatenate([H, X], axis=1)
    W1d = W1_ref[0:32, :] - W1_ref[32:64, :]
    P_ref[...] = jnp.dot(h, W1d, preferred_element_type=jnp.float32) + b1_ref[...]
    Q_ref[...] = jnp.dot(h, W1_ref[32:64, :], preferred_element_type=jnp.float32)


@jax.jit
def _tc_pre(x, W_in, b_in, W1, b1):
    return pl.pallas_call(
        _tc_pre_body,
        grid=(N // NB,),
        in_specs=[
            pl.BlockSpec((NB, 16), lambda i: (i, 0)),
            pl.BlockSpec((16, 16), lambda i: (0, 0)),
            pl.BlockSpec((16,), lambda i: (0,)),
            pl.BlockSpec((64, 16), lambda i: (0, 0)),
            pl.BlockSpec((16,), lambda i: (0,)),
        ],
        out_specs=[
            pl.BlockSpec((NB, 16), lambda i: (i, 0)),
            pl.BlockSpec((NB, 16), lambda i: (i, 0)),
        ],
        out_shape=[
            jax.ShapeDtypeStruct((N, 16), jnp.float32),
            jax.ShapeDtypeStruct((N, 16), jnp.float32),
        ],
    )(x, W_in, b_in, W1, b1)


def _tc_edge_body(gp_ref, gq_ref, W2b_ref, b2b_ref, m_ref):
    t = jax.nn.sigmoid(gp_ref[...] + gq_ref[...])
    m_ref[...] = jax.nn.sigmoid(
        jnp.dot(t, W2b_ref[...], preferred_element_type=jnp.float32)
        + b2b_ref[...])


@jax.jit
def _tc_edge(gp8, gq8, W2big, b2big):
    return pl.pallas_call(
        _tc_edge_body,
        grid=(E8 // EB8,),
        in_specs=[
            pl.BlockSpec((EB8, 128), lambda i: (i, 0)),
            pl.BlockSpec((EB8, 128), lambda i: (i, 0)),
            pl.BlockSpec((128, 128), lambda i: (0, 0)),
            pl.BlockSpec((128,), lambda i: (0,)),
        ],
        out_specs=pl.BlockSpec((EB8, 128), lambda i: (i, 0)),
        out_shape=jax.ShapeDtypeStruct((E8, 128), jnp.float32),
    )(gp8, gq8, W2big, b2big)


def _tc_post_body(part_ref, x_ref, We_ref, be_ref, a_ref, b_ref):
    Hn = part_ref[0] + part_ref[1]
    X = x_ref[...]
    w1 = We_ref[0:16, 0]
    w2 = We_ref[16:32, 0]
    w3 = We_ref[32:48, 0]
    w4 = We_ref[48:64, 0]
    a = (jnp.sum(Hn * w1, axis=1, keepdims=True)
         + jnp.sum(X * w2, axis=1, keepdims=True) + be_ref[0])
    b = (jnp.sum(Hn * w3, axis=1, keepdims=True)
         + jnp.sum(X * w4, axis=1, keepdims=True))
    a_ref[...] = jnp.broadcast_to(a, (NB, 16))
    b_ref[...] = jnp.broadcast_to(b, (NB, 16))


@jax.jit
def _tc_post(part, x, W_e, b_e):
    return pl.pallas_call(
        _tc_post_body,
        grid=(N // NB,),
        in_specs=[
            pl.BlockSpec((2, NB, 16), lambda i: (0, i, 0)),
            pl.BlockSpec((NB, 16), lambda i: (i, 0)),
            pl.BlockSpec((64, 1), lambda i: (0, 0)),
            pl.BlockSpec((1,), lambda i: (0,)),
        ],
        out_specs=[
            pl.BlockSpec((NB, 16), lambda i: (i, 0)),
            pl.BlockSpec((NB, 16), lambda i: (i, 0)),
        ],
        out_shape=[
            jax.ShapeDtypeStruct((N, 16), jnp.float32),
            jax.ShapeDtypeStruct((N, 16), jnp.float32),
        ],
    )(part, x, W_e, b_e)


def _tc_out_body(ga_ref, gb_ref, sel_ref, o_ref):
    s = jax.nn.sigmoid(ga_ref[...] + gb_ref[...])
    o_ref[...] = jnp.dot(s, sel_ref[...], preferred_element_type=jnp.float32)


@jax.jit
def _tc_out(ga8, gb8, sel):
    out = pl.pallas_call(
        _tc_out_body,
        grid=(E8 // EB8,),
        in_specs=[
            pl.BlockSpec((EB8, 128), lambda i: (i, 0)),
            pl.BlockSpec((EB8, 128), lambda i: (i, 0)),
            pl.BlockSpec((128, 8), lambda i: (0, 0)),
        ],
        out_specs=pl.BlockSpec((EB8, 8), lambda i: (i, 0)),
        out_shape=jax.ShapeDtypeStruct((E8, 8), jnp.float32),
    )(ga8, gb8, sel)
    return out.reshape(E)


def kernel(x, edge_index, W_in, b_in, W1, b1, W2, b2, W_e, b_e):
    src = edge_index[0]
    dst = edge_index[1]
    W2big = jnp.kron(jnp.eye(8, dtype=jnp.float32), W2)
    b2big = jnp.tile(b2, 8)
    sel = (jnp.arange(128)[:, None] == jnp.arange(8)[None, :] * 16
           ).astype(jnp.float32)

    P, Q = _tc_pre(x, W_in, b_in, W1, b1)
    gp, gq = _sc_gather_pq(P, Q, dst, src)
    m8 = _tc_edge(gp.reshape(E8, 128), gq.reshape(E8, 128), W2big, b2big)
    part = _sc_scatter_add(m8.reshape(E, 16), dst)
    a, b = _tc_post(part, x, W_e, b_e)
    gb, ga = _sc_gather_pq(b, a, dst, src)
    return _tc_out(ga.reshape(E8, 128), gb.reshape(E8, 128), sel)


# speedup vs baseline: 1.4672x; 1.4672x over previous
import functools

import jax
import jax.numpy as jnp
from jax import lax
from jax.experimental import pallas as pl
from jax.experimental.pallas import tpu as pltpu
from jax.experimental.pallas import tpu_sc as plsc

N = 100000
E = 1600000
E8 = E // 8
NC = 2
NS = 16
NW = NC * NS
EPW = E // NW
CHUNK = 2000
NCHUNK = EPW // CHUNK
SCHUNK = 1000
NSCHUNK = EPW // SCHUNK
STRIPE = 6256
LAST_STRIPE = N - 15 * STRIPE

_mesh = plsc.VectorSubcoreMesh(core_axis_name="c", subcore_axis_name="s")
_sc_params = pltpu.CompilerParams(use_tc_tiling_on_sc=False)


def _wid():
    return lax.axis_index("s") * NC + lax.axis_index("c")


@jax.jit
def _sc_gather_pq(P, Q, dst, src):
    @functools.partial(
        pl.kernel,
        out_type=(
            jax.ShapeDtypeStruct((E, 16), jnp.float32),
            jax.ShapeDtypeStruct((E, 16), jnp.float32),
        ),
        mesh=_mesh,
        compiler_params=_sc_params,
        scratch_types=[
            pltpu.VMEM((CHUNK,), jnp.int32),
            pltpu.VMEM((CHUNK,), jnp.int32),
            pltpu.VMEM((CHUNK, 16), jnp.float32),
            pltpu.VMEM((CHUNK, 16), jnp.float32),
        ],
    )
    def k(P_hbm, Q_hbm, dst_hbm, src_hbm, gp_hbm, gq_hbm,
          idxd_v, idxs_v, rp_v, rq_v):
        base = _wid() * EPW

        @pl.loop(0, NCHUNK)
        def _(c):
            off = base + c * CHUNK
            pltpu.sync_copy(dst_hbm.at[pl.ds(off, CHUNK)], idxd_v)
            pltpu.sync_copy(src_hbm.at[pl.ds(off, CHUNK)], idxs_v)
            pltpu.sync_copy(P_hbm.at[idxd_v], rp_v)
            pltpu.sync_copy(Q_hbm.at[idxs_v], rq_v)
            pltpu.sync_copy(rp_v, gp_hbm.at[pl.ds(off, CHUNK)])
            pltpu.sync_copy(rq_v, gq_hbm.at[pl.ds(off, CHUNK)])

    return k(P, Q, dst, src)


@jax.jit
def _sc_scatter_add(m, dst):
    @functools.partial(
        pl.kernel,
        out_type=jax.ShapeDtypeStruct((NC, N, 16), jnp.float32),
        mesh=_mesh,
        compiler_params=_sc_params,
        scratch_types=[
            pltpu.VMEM((SCHUNK,), jnp.int32),
            pltpu.VMEM((SCHUNK, 16), jnp.float32),
            pltpu.VMEM_SHARED((N, 16), jnp.float32),
        ],
    )
    def k(m_hbm, dst_hbm, out_hbm, idx_v, rows_v, acc_sh):
        cid = lax.axis_index("c")
        sid = lax.axis_index("s")
        base = _wid() * EPW

        s0 = sid * STRIPE

        @pl.loop(0, SCHUNK)
        def _(i):
            rows_v[i, :] = jnp.zeros((16,), jnp.float32)

        @pl.when(sid < 15)
        def _():
            for j in range(STRIPE // SCHUNK):
                pltpu.sync_copy(rows_v,
                                acc_sh.at[pl.ds(s0 + j * SCHUNK, SCHUNK)])
            pltpu.sync_copy(
                rows_v.at[pl.ds(0, STRIPE % SCHUNK)],
                acc_sh.at[pl.ds(s0 + (STRIPE // SCHUNK) * SCHUNK,
                                STRIPE % SCHUNK)])

        @pl.when(sid == 15)
        def _():
            for j in range(LAST_STRIPE // SCHUNK):
                pltpu.sync_copy(rows_v,
                                acc_sh.at[pl.ds(s0 + j * SCHUNK, SCHUNK)])
            pltpu.sync_copy(
                rows_v.at[pl.ds(0, LAST_STRIPE % SCHUNK)],
                acc_sh.at[pl.ds(s0 + (LAST_STRIPE // SCHUNK) * SCHUNK,
                                LAST_STRIPE % SCHUNK)])

        plsc.subcore_barrier()

        @pl.loop(0, NSCHUNK)
        def _(c):
            off = base + c * SCHUNK
            pltpu.sync_copy(dst_hbm.at[pl.ds(off, SCHUNK)], idx_v)
            pltpu.sync_copy(m_hbm.at[pl.ds(off, SCHUNK)], rows_v)
            pltpu.sync_copy(rows_v, acc_sh.at[idx_v], add=True)

        plsc.subcore_barrier()

        @pl.when(sid < 15)
        def _():
            pltpu.sync_copy(acc_sh.at[pl.ds(s0, STRIPE)],
                            out_hbm.at[cid, pl.ds(s0, STRIPE)])

        @pl.when(sid == 15)
        def _():
            pltpu.sync_copy(acc_sh.at[pl.ds(s0, LAST_STRIPE)],
                            out_hbm.at[cid, pl.ds(s0, LAST_STRIPE)])

    return k(m, dst)


NB = 800
EB8 = 2000


def _tc_pre_body(x_ref, Winb_ref, binb_ref, K1_ref, K2_ref, b1b_ref,
                 K3_ref, K4_ref, P_ref, Q_ref):
    X8 = x_ref[...]
    H8 = jnp.tanh(jnp.dot(X8, Winb_ref[...],
                          preferred_element_type=jnp.float32) + binb_ref[...])
    P_ref[...] = (jnp.dot(H8, K1_ref[...], preferred_element_type=jnp.float32)
                  + jnp.dot(X8, K2_ref[...], preferred_element_type=jnp.float32)
                  + b1b_ref[...])
    Q_ref[...] = (jnp.dot(H8, K3_ref[...], preferred_element_type=jnp.float32)
                  + jnp.dot(X8, K4_ref[...], preferred_element_type=jnp.float32))


N8 = N // 8
NB8 = 1000


@jax.jit
def _tc_pre(x8, Winb, binb, K1, K2, b1b, K3, K4):
    wspec = pl.BlockSpec((128, 128), lambda i: (0, 0))
    bspec = pl.BlockSpec((128,), lambda i: (0,))
    nspec = pl.BlockSpec((NB8, 128), lambda i: (i, 0))
    return pl.pallas_call(
        _tc_pre_body,
        grid=((N8 + NB8 - 1) // NB8,),
        in_specs=[nspec, wspec, bspec, wspec, wspec, bspec, wspec, wspec],
        out_specs=[nspec, nspec],
        out_shape=[
            jax.ShapeDtypeStruct((N8, 128), jnp.float32),
            jax.ShapeDtypeStruct((N8, 128), jnp.float32),
        ],
    )(x8, Winb, binb, K1, K2, b1b, K3, K4)


def _tc_edge_body(gp_ref, gq_ref, W2b_ref, b2b_ref, m_ref):
    t = jax.nn.sigmoid(gp_ref[...] + gq_ref[...])
    m_ref[...] = jax.nn.sigmoid(
        jnp.dot(t, W2b_ref[...], preferred_element_type=jnp.float32)
        + b2b_ref[...])


@jax.jit
def _tc_edge(gp8, gq8, W2big, b2big):
    return pl.pallas_call(
        _tc_edge_body,
        grid=(E8 // EB8,),
        in_specs=[
            pl.BlockSpec((EB8, 128), lambda i: (i, 0)),
            pl.BlockSpec((EB8, 128), lambda i: (i, 0)),
            pl.BlockSpec((128, 128), lambda i: (0, 0)),
            pl.BlockSpec((128,), lambda i: (0,)),
        ],
        out_specs=pl.BlockSpec((EB8, 128), lambda i: (i, 0)),
        out_shape=jax.ShapeDtypeStruct((E8, 128), jnp.float32),
    )(gp8, gq8, W2big, b2big)


def _tc_post_body(part_ref, x_ref, A1_ref, A2_ref, A3_ref, A4_ref, be_ref,
                  a_ref, b_ref):
    Hn8 = part_ref[0] + part_ref[1]
    X8 = x_ref[...]
    a_ref[...] = (jnp.dot(Hn8, A1_ref[...], preferred_element_type=jnp.float32)
                  + jnp.dot(X8, A2_ref[...], preferred_element_type=jnp.float32)
                  + be_ref[0])
    b_ref[...] = (jnp.dot(Hn8, A3_ref[...], preferred_element_type=jnp.float32)
                  + jnp.dot(X8, A4_ref[...], preferred_element_type=jnp.float32))


@jax.jit
def _tc_post(part8, x8, A1, A2, A3, A4, b_e):
    wspec = pl.BlockSpec((128, 128), lambda i: (0, 0))
    nspec = pl.BlockSpec((NB8, 128), lambda i: (i, 0))
    return pl.pallas_call(
        _tc_post_body,
        grid=((N8 + NB8 - 1) // NB8,),
        in_specs=[
            pl.BlockSpec((2, NB8, 128), lambda i: (0, i, 0)),
            nspec, wspec, wspec, wspec, wspec,
            pl.BlockSpec((1,), lambda i: (0,)),
        ],
        out_specs=[nspec, nspec],
        out_shape=[
            jax.ShapeDtypeStruct((N8, 128), jnp.float32),
            jax.ShapeDtypeStruct((N8, 128), jnp.float32),
        ],
    )(part8, x8, A1, A2, A3, A4, b_e)


def _tc_out_body(ga_ref, gb_ref, sel_ref, o_ref):
    s = jax.nn.sigmoid(ga_ref[...] + gb_ref[...])
    o_ref[...] = jnp.dot(s, sel_ref[...], preferred_element_type=jnp.float32)


@jax.jit
def _tc_out(ga8, gb8, sel):
    out = pl.pallas_call(
        _tc_out_body,
        grid=(E8 // EB8,),
        in_specs=[
            pl.BlockSpec((EB8, 128), lambda i: (i, 0)),
            pl.BlockSpec((EB8, 128), lambda i: (i, 0)),
            pl.BlockSpec((128, 8), lambda i: (0, 0)),
        ],
        out_specs=pl.BlockSpec((EB8, 8), lambda i: (i, 0)),
        out_shape=jax.ShapeDtypeStruct((E8, 8), jnp.float32),
    )(ga8, gb8, sel)
    return out.reshape(E)


def kernel(x, edge_index, W_in, b_in, W1, b1, W2, b2, W_e, b_e):
    src = edge_index[0]
    dst = edge_index[1]
    eye8 = jnp.eye(8, dtype=jnp.float32)
    ones16 = jnp.ones((16,), jnp.float32)
    W2big = jnp.kron(eye8, W2)
    b2big = jnp.tile(b2, 8)
    sel = (jnp.arange(128)[:, None] == jnp.arange(8)[None, :] * 16
           ).astype(jnp.float32)
    Winb = jnp.kron(eye8, W_in)
    binb = jnp.tile(b_in, 8)
    W1d = W1[0:32] - W1[32:64]
    K1 = jnp.kron(eye8, W1d[:16])
    K2 = jnp.kron(eye8, W1d[16:])
    K3 = jnp.kron(eye8, W1[32:48])
    K4 = jnp.kron(eye8, W1[48:64])
    b1b = jnp.tile(b1, 8)
    A1 = jnp.kron(eye8, jnp.outer(W_e[0:16, 0], ones16))
    A2 = jnp.kron(eye8, jnp.outer(W_e[16:32, 0], ones16))
    A3 = jnp.kron(eye8, jnp.outer(W_e[32:48, 0], ones16))
    A4 = jnp.kron(eye8, jnp.outer(W_e[48:64, 0], ones16))

    x8 = x.reshape(N8, 128)

    P8, Q8 = _tc_pre(x8, Winb, binb, K1, K2, b1b, K3, K4)
    gp, gq = _sc_gather_pq(P8.reshape(N, 16), Q8.reshape(N, 16), dst, src)
    m8 = _tc_edge(gp.reshape(E8, 128), gq.reshape(E8, 128), W2big, b2big)
    part = _sc_scatter_add(m8.reshape(E, 16), dst)
    a8, b8 = _tc_post(part.reshape(NC, N8, 128), x8, A1, A2, A3, A4, b_e)
    gb, ga = _sc_gather_pq(b8.reshape(N, 16), a8.reshape(N, 16), dst, src)
    return _tc_out(ga.reshape(E8, 128), gb.reshape(E8, 128), sel)


# speedup vs baseline: 1.6529x; 1.1266x over previous
import functools

import jax
import jax.numpy as jnp
from jax import lax
from jax.experimental import pallas as pl
from jax.experimental.pallas import tpu as pltpu
from jax.experimental.pallas import tpu_sc as plsc

N = 100000
E = 1600000
E8 = E // 8
NC = 2
NS = 16
NW = NC * NS
EPW = E // NW
CHUNK = 2000
NCHUNK = EPW // CHUNK
SCHUNK = 1000
NSCHUNK = EPW // SCHUNK
STRIPE = 6256
LAST_STRIPE = N - 15 * STRIPE

_mesh = plsc.VectorSubcoreMesh(core_axis_name="c", subcore_axis_name="s")
_sc_params = pltpu.CompilerParams(use_tc_tiling_on_sc=False)


def _wid():
    return lax.axis_index("s") * NC + lax.axis_index("c")


@jax.jit
def _sc_gather_pq(P, Q, ei_flat):
    @functools.partial(
        pl.kernel,
        out_type=(
            jax.ShapeDtypeStruct((E, 16), jnp.float32),
            jax.ShapeDtypeStruct((E, 16), jnp.float32),
        ),
        mesh=_mesh,
        compiler_params=_sc_params,
        scratch_types=[
            pltpu.VMEM((CHUNK,), jnp.int32),
            pltpu.VMEM((CHUNK,), jnp.int32),
            pltpu.VMEM((CHUNK, 16), jnp.float32),
            pltpu.VMEM((CHUNK, 16), jnp.float32),
            pltpu.SemaphoreType.DMA,
            pltpu.SemaphoreType.DMA,
            pltpu.SemaphoreType.DMA,
        ],
    )
    def k(P_hbm, Q_hbm, ei_hbm, gp_hbm, gq_hbm,
          idxd_v, idxs_v, rp_v, rq_v, sem_i, sem_g, sem_w):
        base = _wid() * EPW

        @pl.loop(0, NCHUNK)
        def _(c):
            off = base + c * CHUNK
            @pl.when(c > 0)
            def _():
                pltpu.make_async_copy(
                    rp_v, gp_hbm.at[pl.ds(off, CHUNK)], sem_w).wait()
                pltpu.make_async_copy(
                    rq_v, gq_hbm.at[pl.ds(off, CHUNK)], sem_w).wait()

            ci = pltpu.make_async_copy(
                ei_hbm.at[pl.ds(E + off, CHUNK)], idxd_v, sem_i)
            cs = pltpu.make_async_copy(
                ei_hbm.at[pl.ds(off, CHUNK)], idxs_v, sem_i)
            ci.start()
            cs.start()
            ci.wait()
            cs.wait()
            g1 = pltpu.make_async_copy(P_hbm.at[idxd_v], rp_v, sem_g)
            g2 = pltpu.make_async_copy(Q_hbm.at[idxs_v], rq_v, sem_g)
            g1.start()
            g2.start()
            g1.wait()
            g2.wait()
            pltpu.make_async_copy(
                rp_v, gp_hbm.at[pl.ds(off, CHUNK)], sem_w).start()
            pltpu.make_async_copy(
                rq_v, gq_hbm.at[pl.ds(off, CHUNK)], sem_w).start()

        lastoff = base + (NCHUNK - 1) * CHUNK
        pltpu.make_async_copy(
            rp_v, gp_hbm.at[pl.ds(lastoff, CHUNK)], sem_w).wait()
        pltpu.make_async_copy(
            rq_v, gq_hbm.at[pl.ds(lastoff, CHUNK)], sem_w).wait()

    return k(P, Q, ei_flat)


@jax.jit
def _sc_scatter_add(m, ei_flat):
    @functools.partial(
        pl.kernel,
        out_type=jax.ShapeDtypeStruct((NC, N, 16), jnp.float32),
        mesh=_mesh,
        compiler_params=_sc_params,
        scratch_types=[
            pltpu.VMEM((SCHUNK,), jnp.int32),
            pltpu.VMEM((SCHUNK, 16), jnp.float32),
            pltpu.VMEM_SHARED((N, 16), jnp.float32),
            pltpu.SemaphoreType.DMA,
        ],
    )
    def k(m_hbm, ei_hbm, out_hbm, idx_v, rows_v, acc_sh, sem_l):
        cid = lax.axis_index("c")
        sid = lax.axis_index("s")
        base = _wid() * EPW

        s0 = sid * STRIPE

        @pl.loop(0, SCHUNK)
        def _(i):
            rows_v[i, :] = jnp.zeros((16,), jnp.float32)

        @pl.when(sid < 15)
        def _():
            for j in range(STRIPE // SCHUNK):
                pltpu.sync_copy(rows_v,
                                acc_sh.at[pl.ds(s0 + j * SCHUNK, SCHUNK)])
            pltpu.sync_copy(
                rows_v.at[pl.ds(0, STRIPE % SCHUNK)],
                acc_sh.at[pl.ds(s0 + (STRIPE // SCHUNK) * SCHUNK,
                                STRIPE % SCHUNK)])

        @pl.when(sid == 15)
        def _():
            for j in range(LAST_STRIPE // SCHUNK):
                pltpu.sync_copy(rows_v,
                                acc_sh.at[pl.ds(s0 + j * SCHUNK, SCHUNK)])
            pltpu.sync_copy(
                rows_v.at[pl.ds(0, LAST_STRIPE % SCHUNK)],
                acc_sh.at[pl.ds(s0 + (LAST_STRIPE // SCHUNK) * SCHUNK,
                                LAST_STRIPE % SCHUNK)])

        plsc.subcore_barrier()

        @pl.loop(0, NSCHUNK)
        def _(c):
            off = base + c * SCHUNK
            l1 = pltpu.make_async_copy(
                ei_hbm.at[pl.ds(E + off, SCHUNK)], idx_v, sem_l)
            l2 = pltpu.make_async_copy(
                m_hbm.at[pl.ds(off, SCHUNK)], rows_v, sem_l)
            l1.start()
            l2.start()
            l1.wait()
            l2.wait()
            pltpu.sync_copy(rows_v, acc_sh.at[idx_v], add=True)

        plsc.subcore_barrier()

        @pl.when(sid < 15)
        def _():
            pltpu.sync_copy(acc_sh.at[pl.ds(s0, STRIPE)],
                            out_hbm.at[cid, pl.ds(s0, STRIPE)])

        @pl.when(sid == 15)
        def _():
            pltpu.sync_copy(acc_sh.at[pl.ds(s0, LAST_STRIPE)],
                            out_hbm.at[cid, pl.ds(s0, LAST_STRIPE)])

    return k(m, ei_flat)


NB = 800
EB8 = 2000


def _tc_pre_body(x_ref, Winb_ref, binb_ref, K1_ref, K2_ref, b1b_ref,
                 K3_ref, K4_ref, P_ref, Q_ref):
    X8 = x_ref[...]
    H8 = jnp.tanh(jnp.dot(X8, Winb_ref[...],
                          preferred_element_type=jnp.float32) + binb_ref[...])
    P_ref[...] = (jnp.dot(H8, K1_ref[...], preferred_element_type=jnp.float32)
                  + jnp.dot(X8, K2_ref[...], preferred_element_type=jnp.float32)
                  + b1b_ref[...])
    Q_ref[...] = (jnp.dot(H8, K3_ref[...], preferred_element_type=jnp.float32)
                  + jnp.dot(X8, K4_ref[...], preferred_element_type=jnp.float32))


N8 = N // 8
NB8 = 1000


@jax.jit
def _tc_pre(x8, Winb, binb, K1, K2, b1b, K3, K4):
    wspec = pl.BlockSpec((128, 128), lambda i: (0, 0))
    bspec = pl.BlockSpec((128,), lambda i: (0,))
    nspec = pl.BlockSpec((NB8, 128), lambda i: (i, 0))
    return pl.pallas_call(
        _tc_pre_body,
        grid=((N8 + NB8 - 1) // NB8,),
        in_specs=[nspec, wspec, bspec, wspec, wspec, bspec, wspec, wspec],
        out_specs=[nspec, nspec],
        out_shape=[
            jax.ShapeDtypeStruct((N8, 128), jnp.float32),
            jax.ShapeDtypeStruct((N8, 128), jnp.float32),
        ],
    )(x8, Winb, binb, K1, K2, b1b, K3, K4)


def _tc_edge_body(gp_ref, gq_ref, W2b_ref, b2b_ref, m_ref):
    t = jax.nn.sigmoid(gp_ref[...] + gq_ref[...])
    m_ref[...] = jax.nn.sigmoid(
        jnp.dot(t, W2b_ref[...], preferred_element_type=jnp.float32)
        + b2b_ref[...])


@jax.jit
def _tc_edge(gp8, gq8, W2big, b2big):
    return pl.pallas_call(
        _tc_edge_body,
        grid=(E8 // EB8,),
        in_specs=[
            pl.BlockSpec((EB8, 128), lambda i: (i, 0)),
            pl.BlockSpec((EB8, 128), lambda i: (i, 0)),
            pl.BlockSpec((128, 128), lambda i: (0, 0)),
            pl.BlockSpec((128,), lambda i: (0,)),
        ],
        out_specs=pl.BlockSpec((EB8, 128), lambda i: (i, 0)),
        out_shape=jax.ShapeDtypeStruct((E8, 128), jnp.float32),
    )(gp8, gq8, W2big, b2big)


def _tc_post_body(part_ref, x_ref, A1_ref, A2_ref, A3_ref, A4_ref, be_ref,
                  a_ref, b_ref):
    Hn8 = part_ref[0] + part_ref[1]
    X8 = x_ref[...]
    a_ref[...] = (jnp.dot(Hn8, A1_ref[...], preferred_element_type=jnp.float32)
                  + jnp.dot(X8, A2_ref[...], preferred_element_type=jnp.float32)
                  + be_ref[0])
    b_ref[...] = (jnp.dot(Hn8, A3_ref[...], preferred_element_type=jnp.float32)
                  + jnp.dot(X8, A4_ref[...], preferred_element_type=jnp.float32))


@jax.jit
def _tc_post(part8, x8, A1, A2, A3, A4, b_e):
    wspec = pl.BlockSpec((128, 128), lambda i: (0, 0))
    nspec = pl.BlockSpec((NB8, 128), lambda i: (i, 0))
    return pl.pallas_call(
        _tc_post_body,
        grid=((N8 + NB8 - 1) // NB8,),
        in_specs=[
            pl.BlockSpec((2, NB8, 128), lambda i: (0, i, 0)),
            nspec, wspec, wspec, wspec, wspec,
            pl.BlockSpec((1,), lambda i: (0,)),
        ],
        out_specs=[nspec, nspec],
        out_shape=[
            jax.ShapeDtypeStruct((N8, 128), jnp.float32),
            jax.ShapeDtypeStruct((N8, 128), jnp.float32),
        ],
    )(part8, x8, A1, A2, A3, A4, b_e)


def _tc_out_body(ga_ref, gb_ref, sel_ref, o_ref):
    s = jax.nn.sigmoid(ga_ref[...] + gb_ref[...])
    o_ref[...] = jnp.dot(s, sel_ref[...], preferred_element_type=jnp.float32)


@jax.jit
def _tc_out(ga8, gb8, sel):
    out = pl.pallas_call(
        _tc_out_body,
        grid=(E8 // EB8,),
        in_specs=[
            pl.BlockSpec((EB8, 128), lambda i: (i, 0)),
            pl.BlockSpec((EB8, 128), lambda i: (i, 0)),
            pl.BlockSpec((128, 8), lambda i: (0, 0)),
        ],
        out_specs=pl.BlockSpec((EB8, 8), lambda i: (i, 0)),
        out_shape=jax.ShapeDtypeStruct((E8, 8), jnp.float32),
    )(ga8, gb8, sel)
    return out.reshape(E)


def kernel(x, edge_index, W_in, b_in, W1, b1, W2, b2, W_e, b_e):
    ei_flat = edge_index.reshape(2 * E)
    eye8 = jnp.eye(8, dtype=jnp.float32)
    ones16 = jnp.ones((16,), jnp.float32)
    W2big = jnp.kron(eye8, W2)
    b2big = jnp.tile(b2, 8)
    sel = (jnp.arange(128)[:, None] == jnp.arange(8)[None, :] * 16
           ).astype(jnp.float32)
    Winb = jnp.kron(eye8, W_in)
    binb = jnp.tile(b_in, 8)
    W1d = W1[0:32] - W1[32:64]
    K1 = jnp.kron(eye8, W1d[:16])
    K2 = jnp.kron(eye8, W1d[16:])
    K3 = jnp.kron(eye8, W1[32:48])
    K4 = jnp.kron(eye8, W1[48:64])
    b1b = jnp.tile(b1, 8)
    A1 = jnp.kron(eye8, jnp.outer(W_e[0:16, 0], ones16))
    A2 = jnp.kron(eye8, jnp.outer(W_e[16:32, 0], ones16))
    A3 = jnp.kron(eye8, jnp.outer(W_e[32:48, 0], ones16))
    A4 = jnp.kron(eye8, jnp.outer(W_e[48:64, 0], ones16))

    x8 = x.reshape(N8, 128)

    P8, Q8 = _tc_pre(x8, Winb, binb, K1, K2, b1b, K3, K4)
    gp, gq = _sc_gather_pq(P8.reshape(N, 16), Q8.reshape(N, 16), ei_flat)
    m8 = _tc_edge(gp.reshape(E8, 128), gq.reshape(E8, 128), W2big, b2big)
    part = _sc_scatter_add(m8.reshape(E, 16), ei_flat)
    a8, b8 = _tc_post(part.reshape(NC, N8, 128), x8, A1, A2, A3, A4, b_e)
    gb, ga = _sc_gather_pq(b8.reshape(N, 16), a8.reshape(N, 16), ei_flat)
    return _tc_out(ga.reshape(E8, 128), gb.reshape(E8, 128), sel)
